# Initial kernel scaffold; baseline (speedup 1.0000x reference)
import jax
import jax.numpy as jnp
from jax.experimental import pallas as pl


def kernel(x, W, b):
    raise NotImplementedError("write your pallas kernel here")


# speedup vs baseline: 5.2514x; 5.2514x over previous
import functools

import jax
import jax.numpy as jnp
from jax import lax
from jax.experimental import pallas as pl
from jax.experimental.pallas import tpu as pltpu
from jax.experimental.pallas import tpu_sc as plsc

B = 4
N = 100000
C = 128
K = 10000

NT = 8
BLK = 8192
NBLK = 13
NP = NBLK * BLK
CT = NP // NT
NV = CT // 16

CAP = 1264
LAST_START = K - CAP
NCH = CAP // 16
RND = 5
RCH = 16

INT_MIN = -2147483648
MAXP = 2147483647
NEG_INF = float("-inf")


def _score_body(x_ref, w_ref, bias_ref, o_ref):
  j = pl.program_id(1)
  xb = x_ref[0]
  wv = w_ref[...]
  sc = lax.dot_general(wv, xb, (((1,), (1,)), ((), ())),
                       preferred_element_type=jnp.float32)
  sc = sc + bias_ref[0, 0]
  cols = j * BLK + lax.broadcasted_iota(jnp.int32, (1, BLK), 1)
  o_ref[0, 0] = jnp.where(cols < N, sc, NEG_INF)


def _scores(x, W, bias):
  out4 = pl.pallas_call(
      _score_body,
      grid=(B, NBLK),
      in_specs=[
          pl.BlockSpec((1, BLK, C), lambda i, j: (i, j, 0)),
          pl.BlockSpec((1, C), lambda i, j: (0, 0)),
          pl.BlockSpec((1, 1), lambda i, j: (0, 0)),
      ],
      out_specs=pl.BlockSpec((1, 1, 1, BLK), lambda i, j: (i, j, 0, 0)),
      out_shape=jax.ShapeDtypeStruct((B, NBLK, 1, BLK), jnp.float32),
  )(x, W, bias.reshape(1, 1))
  return out4.reshape(B, NP)


def _keys_at(scores_v, i):
  sv = scores_v[pl.ds(i * 16, 16)]
  bits = lax.bitcast_convert_type(sv, jnp.int32)
  return jnp.where(bits >= 0, bits, bits ^ MAXP)


def _sc_body(scores_hbm, x2_hbm, pooled_hbm, oidx_hbm,
             scores_v, hist, hist8, merged, selbuf, cnt16, tmpa, outbuf,
             idxl, rowbuf, shist, scnt, ssel, sem):
  cc = lax.axis_index("c")
  ss = lax.axis_index("s")
  g = ss // NT
  w = ss % NT
  bb = cc * 2 + g
  base_n = w * CT
  lane = lax.iota(jnp.int32, 16)
  ones = jnp.ones((16,), jnp.int32)

  pltpu.sync_copy(scores_hbm.at[pl.ds(bb * NP + base_n, CT)], scores_v)

  prefix = jnp.int32(0)
  kp = jnp.int32(K)
  for p, shift in enumerate((24, 16, 8, 0)):
    sh = jnp.int32(shift)

    def zbody(t, _):
      hist[pl.ds(t * 16, 16)] = jnp.zeros((16,), jnp.int32)
      return _
    lax.fori_loop(0, 256, zbody, 0)

    def hbody(i, _):
      key = _keys_at(scores_v, i)
      ub = key ^ INT_MIN
      bucket = lax.shift_right_logical(ub, sh) & 255
      if shift == 24:
        m = lane < 16
      else:
        m = lax.shift_right_logical(ub, sh + 8) == prefix
      plsc.addupdate_scatter(hist, [lane * 256 + bucket], ones, mask=m)
      return _
    lax.fori_loop(0, NV, hbody, 0)

    def mbody(jv, _):
      acc = hist[pl.ds(jv * 16, 16)]
      for r in range(1, 16):
        acc = acc + hist[pl.ds(r * 256 + jv * 16, 16)]
      merged[pl.ds(jv * 16, 16)] = acc
      return _
    lax.fori_loop(0, 16, mbody, 0)
    pltpu.sync_copy(merged, shist.at[p, g, w])
    plsc.subcore_barrier()

    pltpu.sync_copy(shist.at[p, g], hist8)
    def gbody(jv, _):
      acc = hist8[0, pl.ds(jv * 16, 16)]
      for r in range(1, NT):
        acc = acc + hist8[r, pl.ds(jv * 16, 16)]
      merged[pl.ds(jv * 16, 16)] = acc
      return _
    lax.fori_loop(0, 16, gbody, 0)

    def cnt_ge(mval):
      def cbody(jv, a):
        vec = merged[pl.ds(jv * 16, 16)]
        bins = jv * 16 + lane
        return a + jnp.sum(jnp.where(bins >= mval, vec, 0))
      return lax.fori_loop(0, 16, cbody, jnp.int32(0))

    def sbody(_, lohi):
      lo, hi = lohi
      mid = (lo + hi + 1) >> 1
      take = cnt_ge(mid) >= kp
      return (jnp.where(take, mid, lo), jnp.where(take, hi, mid - 1))
    lo, _hi = lax.fori_loop(0, 8, sbody, (jnp.int32(0), jnp.int32(255)))
    bstar = lo
    kp = kp - cnt_ge(bstar + 1)
    prefix = (prefix << 8) | bstar

  t_key = prefix ^ INT_MIN
  need_eq = kp

  def ctbody(i, a):
    cgt, ceq = a
    key = _keys_at(scores_v, i)
    cgt = cgt + jnp.sum(jnp.where(key > t_key, 1, 0))
    ceq = ceq + jnp.sum(jnp.where(key == t_key, 1, 0))
    return (cgt, ceq)
  c_gt, c_eq = lax.fori_loop(0, NV, ctbody, (jnp.int32(0), jnp.int32(0)))

  cnt16[...] = jnp.where(lane == 0, c_gt, 0) + jnp.where(lane == 1, c_eq, 0)
  pltpu.sync_copy(cnt16, scnt.at[g, w])
  plsc.subcore_barrier()

  cgt_l, ceq_l = [], []
  for v in range(NT):
    pltpu.sync_copy(scnt.at[g, v], cnt16)
    vec = cnt16[...]
    cgt_l.append(jnp.sum(jnp.where(lane == 0, vec, 0)))
    ceq_l.append(jnp.sum(jnp.where(lane == 1, vec, 0)))

  eqpref = jnp.int32(0)
  off = jnp.int32(0)
  off_l, cw_l = [], []
  for v in range(NT):
    e_v = jnp.clip(need_eq - eqpref, 0, ceq_l[v])
    c_v = cgt_l[v] + e_v
    off_l.append(off)
    cw_l.append(c_v)
    eqpref = eqpref + ceq_l[v]
    off = off + c_v

  my_off = jnp.int32(0)
  my_eqpref = jnp.int32(0)
  for v in range(NT):
    my_off = my_off + jnp.where(w == v, off_l[v], 0)
    my_eqpref = my_eqpref + jnp.where(w > v, ceq_l[v], 0)

  def pbody(i, a):
    nsel, neq = a
    key = _keys_at(scores_v, i)
    m_gt = key > t_key
    m_eq = key == t_key
    meqi = jnp.where(m_eq, 1, 0)
    eqrank = my_eqpref + neq + plsc.cumsum(meqi) - meqi
    m = m_gt | (m_eq & (eqrank < need_eq))
    mi = jnp.where(m, 1, 0)
    pos = nsel + plsc.cumsum(mi) - mi
    nodeidx = base_n + i * 16 + lane
    plsc.store_scatter(selbuf, [pos], nodeidx, mask=m)
    return (nsel + jnp.sum(mi), neq + jnp.sum(meqi))
  lax.fori_loop(0, NV, pbody, (jnp.int32(0), jnp.int32(0)))

  pltpu.sync_copy(selbuf, ssel.at[g, w])
  plsc.subcore_barrier()

  @pl.when(w == 0)
  def _assemble():
    for v in range(NT):
      trips = (cw_l[v] + 1023) >> 10
      base_o = off_l[v]

      def tbody(t, _, v=v, base_o=base_o):
        pltpu.sync_copy(ssel.at[g, v, pl.ds(t * 1024, 1024)], tmpa)
        def ubody(u, __, t=t):
          vec = tmpa[pl.ds(u * 16, 16)]
          dst = base_o + t * 1024 + u * 16 + lane
          plsc.store_scatter(outbuf, [dst], vec, mask=lane < 16)
          return __
        lax.fori_loop(0, 64, ubody, 0)
        return _
      lax.fori_loop(0, trips, tbody, 0)
    pltpu.sync_copy(outbuf.at[pl.ds(0, K)], oidx_hbm.at[pl.ds(bb * K, K)])
  plsc.subcore_barrier()

  a_w = jnp.minimum(w * CAP, jnp.int32(LAST_START))
  pltpu.sync_copy(oidx_hbm.at[pl.ds(bb * K + a_w, CAP)], idxl)
  boff = bb * N
  for r in range(RND):
    nch = min(RCH, NCH - r * RCH)
    descs = []
    for t in range(nch):
      q = r * RCH + t
      iv = idxl[pl.ds(q * 16, 16)] + boff
      descs.append(
          pltpu.async_copy(x2_hbm.at[iv], rowbuf.at[pl.ds(t * 16, 16), :],
                           sem))
    for d in descs:
      d.wait()
    nrows = nch * 16
    pltpu.sync_copy(rowbuf.at[pl.ds(0, nrows), :],
                    pooled_hbm.at[pl.ds(bb * K + a_w + r * RCH * 16, nrows)])


@functools.partial(jax.jit, static_argnames=())
def _run(x, W, bias):
  scores = _scores(x, W, bias)
  x2 = x.reshape(B * N, C)
  sck = pl.kernel(
      _sc_body,
      out_type=(
          jax.ShapeDtypeStruct((B * K, C), jnp.float32),
          jax.ShapeDtypeStruct((B * K,), jnp.int32),
      ),
      mesh=plsc.VectorSubcoreMesh(core_axis_name="c", subcore_axis_name="s"),
      compiler_params=pltpu.CompilerParams(needs_layout_passes=False),
      scratch_types=[
          pltpu.VMEM((CT,), jnp.float32),
          pltpu.VMEM((4096,), jnp.int32),
          pltpu.VMEM((NT, 256), jnp.int32),
          pltpu.VMEM((256,), jnp.int32),
          pltpu.VMEM((CT,), jnp.int32),
          pltpu.VMEM((16,), jnp.int32),
          pltpu.VMEM((1024,), jnp.int32),
          pltpu.VMEM((11072,), jnp.int32),
          pltpu.VMEM((CAP,), jnp.int32),
          pltpu.VMEM((RCH * 16, C), jnp.float32),
          pltpu.VMEM_SHARED((4, 2, NT, 256), jnp.int32),
          pltpu.VMEM_SHARED((2, NT, 16), jnp.int32),
          pltpu.VMEM_SHARED((2, NT, CT), jnp.int32),
          pltpu.SemaphoreType.DMA,
      ],
  )
  pooled, idx = sck(scores.reshape(B * NP), x2)
  return pooled.reshape(B, K, C), idx.reshape(B, K)


def kernel(x, W, b):
  return _run(x, W, b)


# speedup vs baseline: 5.6305x; 1.0722x over previous
import functools

import jax
import jax.numpy as jnp
from jax import lax
from jax.experimental import pallas as pl
from jax.experimental.pallas import tpu as pltpu
from jax.experimental.pallas import tpu_sc as plsc

B = 4
N = 100000
C = 128
K = 10000

NT = 8
BLK = 8192
NBLK = 13
NP = NBLK * BLK
CT = NP // NT
NV = CT // 16

CAP = 1264
LAST_START = K - CAP
NCH = CAP // 16
RCH = 8
RND = 10
SSROW = 10240

INT_MIN = -2147483648
MAXP = 2147483647
NEG_INF = float("-inf")


def _score_body(x_ref, w_ref, bias_ref, o_ref):
  j = pl.program_id(1)
  xb = x_ref[0]
  wv = w_ref[...]
  sc = lax.dot_general(wv, xb, (((1,), (1,)), ((), ())),
                       preferred_element_type=jnp.float32)
  sc = sc + bias_ref[0, 0]
  cols = j * BLK + lax.broadcasted_iota(jnp.int32, (1, BLK), 1)
  o_ref[0, 0] = jnp.where(cols < N, sc, NEG_INF)


def _scores(x, W, bias):
  out4 = pl.pallas_call(
      _score_body,
      grid=(B, NBLK),
      in_specs=[
          pl.BlockSpec((1, BLK, C), lambda i, j: (i, j, 0)),
          pl.BlockSpec((1, C), lambda i, j: (0, 0)),
          pl.BlockSpec((1, 1), lambda i, j: (0, 0)),
      ],
      out_specs=pl.BlockSpec((1, 1, 1, BLK), lambda i, j: (i, j, 0, 0)),
      out_shape=jax.ShapeDtypeStruct((B, NBLK, 1, BLK), jnp.float32),
  )(x, W, bias.reshape(1, 1))
  return out4.reshape(B, NP)


def _keys_at(scores_v, i):
  sv = scores_v[pl.ds(i * 16, 16)]
  bits = lax.bitcast_convert_type(sv, jnp.int32)
  return jnp.where(bits >= 0, bits, bits ^ MAXP)


def _sc_body(scores_hbm, x2_hbm, pooled_hbm, oidx_hbm,
             scores_v, candk, selbuf, hist, hist8, merged, lmerged, cnt16,
             tmpa, outbuf, idxl, rowbufa, rowbufb, shist, scnt, ssel,
             sema, semb):
  cc = lax.axis_index("c")
  ss = lax.axis_index("s")
  g = ss // NT
  w = ss % NT
  bb = cc * 2 + g
  base_n = w * CT
  lane = lax.iota(jnp.int32, 16)
  ones = jnp.ones((16,), jnp.int32)
  alltrue = lane < 16

  pltpu.sync_copy(scores_hbm.at[pl.ds(bb * NP + base_n, CT)], scores_v)

  def zero_hist():
    def zbody(t, _):
      hist[pl.ds(t * 16, 16)] = jnp.zeros((16,), jnp.int32)
      return _
    lax.fori_loop(0, 256, zbody, 0)

  def merge_publish_search(p, kp):
    def mbody(jv, _):
      acc = hist[pl.ds(jv * 16, 16)]
      for r in range(1, 16):
        acc = acc + hist[pl.ds(r * 256 + jv * 16, 16)]
      lmerged[pl.ds(jv * 16, 16)] = acc
      return _
    lax.fori_loop(0, 16, mbody, 0)
    pltpu.sync_copy(lmerged, shist.at[p, g, w])
    plsc.subcore_barrier()
    pltpu.sync_copy(shist.at[p, g], hist8)

    def gbody(jv, _):
      acc = hist8[0, pl.ds(jv * 16, 16)]
      for r in range(1, NT):
        acc = acc + hist8[r, pl.ds(jv * 16, 16)]
      merged[pl.ds(jv * 16, 16)] = acc
      return _
    lax.fori_loop(0, 16, gbody, 0)

    def cnt_ge(ref, mval):
      def cbody(jv, a):
        vec = ref[pl.ds(jv * 16, 16)]
        bins = jv * 16 + lane
        return a + jnp.sum(jnp.where(bins >= mval, vec, 0))
      return lax.fori_loop(0, 16, cbody, jnp.int32(0))

    def sbody(_, lohi):
      lo, hi = lohi
      mid = (lo + hi + 1) >> 1
      take = cnt_ge(merged, mid) >= kp
      return (jnp.where(take, mid, lo), jnp.where(take, hi, mid - 1))
    bstar, _hi = lax.fori_loop(0, 8, sbody, (jnp.int32(0), jnp.int32(255)))
    cnt_above = cnt_ge(merged, bstar + 1)
    loc_above = cnt_ge(lmerged, bstar + 1)
    loc_eq = cnt_ge(lmerged, bstar) - loc_above
    return bstar, cnt_above, loc_above, loc_eq

  zero_hist()

  def h1body(i, _):
    ub = _keys_at(scores_v, i) ^ INT_MIN
    bucket = lax.shift_right_logical(ub, jnp.int32(24))
    plsc.addupdate_scatter(hist, [lane * 256 + bucket], ones, mask=alltrue)
    return _
  lax.fori_loop(0, NV, h1body, 0)

  kp = jnp.int32(K)
  b1, ca1, la1, _le1 = merge_publish_search(0, kp)
  kp = kp - ca1
  prefix = b1
  c_gt_local = la1

  zero_hist()

  def h2body(i, nc):
    ub = _keys_at(scores_v, i) ^ INT_MIN
    m = lax.shift_right_logical(ub, jnp.int32(24)) == prefix
    bucket = lax.shift_right_logical(ub, jnp.int32(16)) & 255
    plsc.addupdate_scatter(hist, [lane * 256 + bucket], ones, mask=m)
    mi = jnp.where(m, 1, 0)
    pos = nc + plsc.cumsum(mi) - mi
    plsc.store_scatter(candk, [pos], ub, mask=m)
    return nc + jnp.sum(mi)
  nc = lax.fori_loop(0, NV, h2body, jnp.int32(0))
  ncv = (nc + 15) >> 4

  b2, ca2, la2, _le2 = merge_publish_search(1, kp)
  kp = kp - ca2
  prefix = (prefix << 8) | b2
  c_gt_local = c_gt_local + la2

  zero_hist()

  def h3body(i, _):
    ub = candk[pl.ds(i * 16, 16)]
    valid = (i * 16 + lane) < nc
    m = valid & (lax.shift_right_logical(ub, jnp.int32(16)) == prefix)
    bucket = lax.shift_right_logical(ub, jnp.int32(8)) & 255
    plsc.addupdate_scatter(hist, [lane * 256 + bucket], ones, mask=m)
    return _
  lax.fori_loop(0, ncv, h3body, 0)

  b3, ca3, la3, _le3 = merge_publish_search(2, kp)
  kp = kp - ca3
  prefix = (prefix << 8) | b3
  c_gt_local = c_gt_local + la3

  zero_hist()

  def h4body(i, _):
    ub = candk[pl.ds(i * 16, 16)]
    valid = (i * 16 + lane) < nc
    m = valid & (lax.shift_right_logical(ub, jnp.int32(8)) == prefix)
    bucket = ub & 255
    plsc.addupdate_scatter(hist, [lane * 256 + bucket], ones, mask=m)
    return _
  lax.fori_loop(0, ncv, h4body, 0)

  b4, ca4, la4, le4 = merge_publish_search(3, kp)
  kp = kp - ca4
  prefix = (prefix << 8) | b4
  c_gt = c_gt_local + la4
  c_eq = le4

  t_key = prefix ^ INT_MIN
  need_eq = kp

  cnt16[...] = jnp.where(lane == 0, c_gt, 0) + jnp.where(lane == 1, c_eq, 0)
  pltpu.sync_copy(cnt16, scnt.at[g, w])
  plsc.subcore_barrier()

  cgt_l, ceq_l = [], []
  for v in range(NT):
    pltpu.sync_copy(scnt.at[g, v], cnt16)
    vec = cnt16[...]
    cgt_l.append(jnp.sum(jnp.where(lane == 0, vec, 0)))
    ceq_l.append(jnp.sum(jnp.where(lane == 1, vec, 0)))

  eqpref = jnp.int32(0)
  off = jnp.int32(0)
  off_l, cw_l = [], []
  for v in range(NT):
    e_v = jnp.clip(need_eq - eqpref, 0, ceq_l[v])
    c_v = cgt_l[v] + e_v
    off_l.append(off)
    cw_l.append(c_v)
    eqpref = eqpref + ceq_l[v]
    off = off + c_v

  my_eqpref = jnp.int32(0)
  for v in range(NT):
    my_eqpref = my_eqpref + jnp.where(w > v, ceq_l[v], 0)

  def pbody(i, a):
    nsel, neq = a
    key = _keys_at(scores_v, i)
    m_gt = key > t_key
    m_eq = key == t_key
    meqi = jnp.where(m_eq, 1, 0)
    eqrank = my_eqpref + neq + plsc.cumsum(meqi) - meqi
    m = m_gt | (m_eq & (eqrank < need_eq))
    mi = jnp.where(m, 1, 0)
    pos = nsel + plsc.cumsum(mi) - mi
    nodeidx = base_n + i * 16 + lane
    plsc.store_scatter(selbuf, [pos], nodeidx, mask=m)
    return (nsel + jnp.sum(mi), neq + jnp.sum(meqi))
  lax.fori_loop(0, NV, pbody, (jnp.int32(0), jnp.int32(0)))

  pltpu.sync_copy(selbuf, ssel.at[g, w])
  plsc.subcore_barrier()

  @pl.when(w == 0)
  def _assemble():
    for v in range(NT):
      trips = (cw_l[v] + 1023) >> 10
      base_o = off_l[v]

      def tbody(t, _, v=v, base_o=base_o):
        pltpu.sync_copy(ssel.at[g, v, pl.ds(t * 1024, 1024)], tmpa)
        def ubody(u, __, t=t):
          vec = tmpa[pl.ds(u * 16, 16)]
          dst = base_o + t * 1024 + u * 16 + lane
          plsc.store_scatter(outbuf, [dst], vec, mask=alltrue)
          return __
        lax.fori_loop(0, 64, ubody, 0)
        return _
      lax.fori_loop(0, trips, tbody, 0)
    pltpu.sync_copy(outbuf.at[pl.ds(0, K)], oidx_hbm.at[pl.ds(bb * K, K)])
  plsc.subcore_barrier()

  a_w = jnp.minimum(w * CAP, jnp.int32(LAST_START))
  pltpu.sync_copy(oidx_hbm.at[pl.ds(bb * K + a_w, CAP)], idxl)
  boff = bb * N
  bufs = (rowbufa, rowbufb)
  sems = (sema, semb)
  descs = {}
  nrows_l = {}
  for r in range(RND):
    nch = min(RCH, NCH - r * RCH)
    nrows_l[r] = nch * 16
    buf, sem = bufs[r % 2], sems[r % 2]
    dl = []
    for t in range(nch):
      q = r * RCH + t
      iv = idxl[pl.ds(q * 16, 16)] + boff
      dl.append(
          pltpu.async_copy(x2_hbm.at[iv], buf.at[pl.ds(t * 16, 16), :], sem))
    descs[r] = dl
    if r > 0:
      for d in descs[r - 1]:
        d.wait()
      pbuf = bufs[(r - 1) % 2]
      pltpu.sync_copy(
          pbuf.at[pl.ds(0, nrows_l[r - 1]), :],
          pooled_hbm.at[pl.ds(bb * K + a_w + (r - 1) * RCH * 16,
                              nrows_l[r - 1])])
  r = RND - 1
  for d in descs[r]:
    d.wait()
  pltpu.sync_copy(
      bufs[r % 2].at[pl.ds(0, nrows_l[r]), :],
      pooled_hbm.at[pl.ds(bb * K + a_w + r * RCH * 16, nrows_l[r])])


@functools.partial(jax.jit, static_argnames=())
def _run(x, W, bias):
  scores = _scores(x, W, bias)
  x2 = x.reshape(B * N, C)
  sck = pl.kernel(
      _sc_body,
      out_type=(
          jax.ShapeDtypeStruct((B * K, C), jnp.float32),
          jax.ShapeDtypeStruct((B * K,), jnp.int32),
      ),
      mesh=plsc.VectorSubcoreMesh(core_axis_name="c", subcore_axis_name="s"),
      compiler_params=pltpu.CompilerParams(needs_layout_passes=False),
      scratch_types=[
          pltpu.VMEM((CT,), jnp.float32),
          pltpu.VMEM((CT,), jnp.int32),
          pltpu.VMEM((SSROW,), jnp.int32),
          pltpu.VMEM((4096,), jnp.int32),
          pltpu.VMEM((NT, 256), jnp.int32),
          pltpu.VMEM((256,), jnp.int32),
          pltpu.VMEM((256,), jnp.int32),
          pltpu.VMEM((16,), jnp.int32),
          pltpu.VMEM((1024,), jnp.int32),
          pltpu.VMEM((11072,), jnp.int32),
          pltpu.VMEM((CAP,), jnp.int32),
          pltpu.VMEM((RCH * 16, C), jnp.float32),
          pltpu.VMEM((RCH * 16, C), jnp.float32),
          pltpu.VMEM_SHARED((4, 2, NT, 256), jnp.int32),
          pltpu.VMEM_SHARED((2, NT, 16), jnp.int32),
          pltpu.VMEM_SHARED((2, NT, SSROW), jnp.int32),
          pltpu.SemaphoreType.DMA,
          pltpu.SemaphoreType.DMA,
      ],
  )
  pooled, idx = sck(scores.reshape(B * NP), x2)
  return pooled.reshape(B, K, C), idx.reshape(B, K)


def kernel(x, W, b):
  return _run(x, W, b)


# speedup vs baseline: 5.7052x; 1.0133x over previous
import functools

import jax
import jax.numpy as jnp
from jax import lax
from jax.experimental import pallas as pl
from jax.experimental.pallas import tpu as pltpu
from jax.experimental.pallas import tpu_sc as plsc

B = 4
N = 100000
C = 128
K = 10000

NT = 8
BLK = 16384
NBLK = 7
NP = NBLK * BLK
CT = NP // NT
NV = CT // 16

CAP = 1264
LAST_START = K - CAP
NCH = CAP // 16
GCH = (128,) * 9 + (112,)
SSROW = 10240

INT_MIN = -2147483648
MAXP = 2147483647
NEG_INF = float("-inf")


def _score_body(x_ref, w_ref, bias_ref, o_ref):
  j = pl.program_id(1)
  xb = x_ref[0]
  wv = w_ref[...]
  sc = lax.dot_general(wv, xb, (((1,), (1,)), ((), ())),
                       preferred_element_type=jnp.float32)
  sc = sc + bias_ref[0, 0]
  cols = j * BLK + lax.broadcasted_iota(jnp.int32, (1, BLK), 1)
  o_ref[0, 0] = jnp.where(cols < N, sc, NEG_INF)


def _scores(x, W, bias):
  out4 = pl.pallas_call(
      _score_body,
      grid=(B, NBLK),
      in_specs=[
          pl.BlockSpec((1, BLK, C), lambda i, j: (i, j, 0)),
          pl.BlockSpec((1, C), lambda i, j: (0, 0)),
          pl.BlockSpec((1, 1), lambda i, j: (0, 0)),
      ],
      out_specs=pl.BlockSpec((1, 1, 1, BLK), lambda i, j: (i, j, 0, 0)),
      out_shape=jax.ShapeDtypeStruct((B, NBLK, 1, BLK), jnp.float32),
  )(x, W, bias.reshape(1, 1))
  return out4.reshape(B, NP)


def _keys_at(scores_v, i):
  sv = scores_v[pl.ds(i * 16, 16)]
  bits = lax.bitcast_convert_type(sv, jnp.int32)
  return jnp.where(bits >= 0, bits, bits ^ MAXP)


def _sc_body(scores_hbm, x2_hbm, pooled_hbm, oidx_hbm,
             scores_v, candk, selbuf, hist, hist8, merged, lmerged, cnt16,
             tmpa, outbuf, idxl, rowbufa, rowbufb, shist, scnt, ssel,
             sema, semb):
  cc = lax.axis_index("c")
  ss = lax.axis_index("s")
  g = ss // NT
  w = ss % NT
  bb = cc * 2 + g
  base_n = w * CT
  lane = lax.iota(jnp.int32, 16)
  ones = jnp.ones((16,), jnp.int32)
  alltrue = lane < 16

  pltpu.sync_copy(scores_hbm.at[pl.ds(bb * NP + base_n, CT)], scores_v)

  def zero_hist():
    def zbody(t, _):
      hist[pl.ds(t * 16, 16)] = jnp.zeros((16,), jnp.int32)
      return _
    lax.fori_loop(0, 256, zbody, 0)

  def merge_publish_search(p, kp):
    def mbody(jv, _):
      acc = hist[pl.ds(jv * 16, 16)]
      for r in range(1, 16):
        acc = acc + hist[pl.ds(r * 256 + jv * 16, 16)]
      lmerged[pl.ds(jv * 16, 16)] = acc
      return _
    lax.fori_loop(0, 16, mbody, 0)
    pltpu.sync_copy(lmerged, shist.at[p, g, w])
    plsc.subcore_barrier()
    pltpu.sync_copy(shist.at[p, g], hist8)

    def gbody(jv, _):
      acc = hist8[0, pl.ds(jv * 16, 16)]
      for r in range(1, NT):
        acc = acc + hist8[r, pl.ds(jv * 16, 16)]
      merged[pl.ds(jv * 16, 16)] = acc
      return _
    lax.fori_loop(0, 16, gbody, 0)

    def cnt_ge(ref, mval):
      def cbody(jv, a):
        vec = ref[pl.ds(jv * 16, 16)]
        bins = jv * 16 + lane
        return a + jnp.sum(jnp.where(bins >= mval, vec, 0))
      return lax.fori_loop(0, 16, cbody, jnp.int32(0))

    def sbody(_, lohi):
      lo, hi = lohi
      mid = (lo + hi + 1) >> 1
      take = cnt_ge(merged, mid) >= kp
      return (jnp.where(take, mid, lo), jnp.where(take, hi, mid - 1))
    bstar, _hi = lax.fori_loop(0, 8, sbody, (jnp.int32(0), jnp.int32(255)))
    cnt_above = cnt_ge(merged, bstar + 1)
    loc_above = cnt_ge(lmerged, bstar + 1)
    loc_eq = cnt_ge(lmerged, bstar) - loc_above
    return bstar, cnt_above, loc_above, loc_eq

  zero_hist()

  def h1body(i, _):
    ub = _keys_at(scores_v, i) ^ INT_MIN
    bucket = lax.shift_right_logical(ub, jnp.int32(24))
    plsc.addupdate_scatter(hist, [lane * 256 + bucket], ones, mask=alltrue)
    return _
  lax.fori_loop(0, NV, h1body, 0)

  kp = jnp.int32(K)
  b1, ca1, la1, _le1 = merge_publish_search(0, kp)
  kp = kp - ca1
  prefix = b1
  c_gt_local = la1

  zero_hist()

  def h2body(i, nc):
    ub = _keys_at(scores_v, i) ^ INT_MIN
    m = lax.shift_right_logical(ub, jnp.int32(24)) == prefix
    bucket = lax.shift_right_logical(ub, jnp.int32(16)) & 255
    plsc.addupdate_scatter(hist, [lane * 256 + bucket], ones, mask=m)
    mi = jnp.where(m, 1, 0)
    pos = nc + plsc.cumsum(mi) - mi
    plsc.store_scatter(candk, [pos], ub, mask=m)
    return nc + jnp.sum(mi)
  nc = lax.fori_loop(0, NV, h2body, jnp.int32(0))
  ncv = (nc + 15) >> 4

  b2, ca2, la2, _le2 = merge_publish_search(1, kp)
  kp = kp - ca2
  prefix = (prefix << 8) | b2
  c_gt_local = c_gt_local + la2

  zero_hist()

  def h3body(i, _):
    ub = candk[pl.ds(i * 16, 16)]
    valid = (i * 16 + lane) < nc
    m = valid & (lax.shift_right_logical(ub, jnp.int32(16)) == prefix)
    bucket = lax.shift_right_logical(ub, jnp.int32(8)) & 255
    plsc.addupdate_scatter(hist, [lane * 256 + bucket], ones, mask=m)
    return _
  lax.fori_loop(0, ncv, h3body, 0)

  b3, ca3, la3, _le3 = merge_publish_search(2, kp)
  kp = kp - ca3
  prefix = (prefix << 8) | b3
  c_gt_local = c_gt_local + la3

  zero_hist()

  def h4body(i, _):
    ub = candk[pl.ds(i * 16, 16)]
    valid = (i * 16 + lane) < nc
    m = valid & (lax.shift_right_logical(ub, jnp.int32(8)) == prefix)
    bucket = ub & 255
    plsc.addupdate_scatter(hist, [lane * 256 + bucket], ones, mask=m)
    return _
  lax.fori_loop(0, ncv, h4body, 0)

  b4, ca4, la4, le4 = merge_publish_search(3, kp)
  kp = kp - ca4
  prefix = (prefix << 8) | b4
  c_gt = c_gt_local + la4
  c_eq = le4

  t_key = prefix ^ INT_MIN
  need_eq = kp

  cnt16[...] = jnp.where(lane == 0, c_gt, 0) + jnp.where(lane == 1, c_eq, 0)
  pltpu.sync_copy(cnt16, scnt.at[g, w])
  plsc.subcore_barrier()

  cgt_l, ceq_l = [], []
  for v in range(NT):
    pltpu.sync_copy(scnt.at[g, v], cnt16)
    vec = cnt16[...]
    cgt_l.append(jnp.sum(jnp.where(lane == 0, vec, 0)))
    ceq_l.append(jnp.sum(jnp.where(lane == 1, vec, 0)))

  eqpref = jnp.int32(0)
  off = jnp.int32(0)
  off_l, cw_l = [], []
  for v in range(NT):
    e_v = jnp.clip(need_eq - eqpref, 0, ceq_l[v])
    c_v = cgt_l[v] + e_v
    off_l.append(off)
    cw_l.append(c_v)
    eqpref = eqpref + ceq_l[v]
    off = off + c_v

  my_eqpref = jnp.int32(0)
  for v in range(NT):
    my_eqpref = my_eqpref + jnp.where(w > v, ceq_l[v], 0)

  def pbody(i, a):
    nsel, neq = a
    key = _keys_at(scores_v, i)
    m_gt = key > t_key
    m_eq = key == t_key
    meqi = jnp.where(m_eq, 1, 0)
    eqrank = my_eqpref + neq + plsc.cumsum(meqi) - meqi
    m = m_gt | (m_eq & (eqrank < need_eq))
    mi = jnp.where(m, 1, 0)
    pos = nsel + plsc.cumsum(mi) - mi
    nodeidx = base_n + i * 16 + lane
    plsc.store_scatter(selbuf, [pos], nodeidx, mask=m)
    return (nsel + jnp.sum(mi), neq + jnp.sum(meqi))
  lax.fori_loop(0, NV, pbody, (jnp.int32(0), jnp.int32(0)))

  pltpu.sync_copy(selbuf, ssel.at[g, w])
  plsc.subcore_barrier()

  @pl.when(w == 0)
  def _assemble():
    for v in range(NT):
      trips = (cw_l[v] + 15) >> 4
      base_o = off_l[v]
      pltpu.sync_copy(ssel.at[g, v], tmpa)

      def ubody(u, __, base_o=base_o):
        vec = tmpa[pl.ds(u * 16, 16)]
        dst = base_o + u * 16 + lane
        plsc.store_scatter(outbuf, [dst], vec, mask=alltrue)
        return __
      lax.fori_loop(0, trips, ubody, 0)
    pltpu.sync_copy(outbuf.at[pl.ds(0, K)], oidx_hbm.at[pl.ds(bb * K, K)])
  plsc.subcore_barrier()

  a_w = jnp.minimum(w * CAP, jnp.int32(LAST_START))
  pltpu.sync_copy(oidx_hbm.at[pl.ds(bb * K + a_w, CAP)], idxl)
  boff = bb * N
  for q in range(NCH):
    idxl[pl.ds(q * 16, 16)] = idxl[pl.ds(q * 16, 16)] + boff
  bufs = (rowbufa, rowbufb)
  sems = (sema, semb)
  prev = None
  off = 0
  for i, sz in enumerate(GCH):
    buf, sem = bufs[i % 2], sems[i % 2]
    d = pltpu.async_copy(x2_hbm.at[idxl.at[pl.ds(off, sz)]],
                         buf.at[pl.ds(0, sz), :], sem)
    if prev is not None:
      pd, poff, psz, pbuf = prev
      pd.wait()
      pltpu.sync_copy(pbuf.at[pl.ds(0, psz), :],
                      pooled_hbm.at[pl.ds(bb * K + a_w + poff, psz)])
    prev = (d, off, sz, buf)
    off += sz
  pd, poff, psz, pbuf = prev
  pd.wait()
  pltpu.sync_copy(pbuf.at[pl.ds(0, psz), :],
                  pooled_hbm.at[pl.ds(bb * K + a_w + poff, psz)])


@functools.partial(jax.jit, static_argnames=())
def _run(x, W, bias):
  scores = _scores(x, W, bias)
  x2 = x.reshape(B * N, C)
  sck = pl.kernel(
      _sc_body,
      out_type=(
          jax.ShapeDtypeStruct((B * K, C), jnp.float32),
          jax.ShapeDtypeStruct((B * K,), jnp.int32),
      ),
      mesh=plsc.VectorSubcoreMesh(core_axis_name="c", subcore_axis_name="s"),
      compiler_params=pltpu.CompilerParams(needs_layout_passes=False),
      scratch_types=[
          pltpu.VMEM((CT,), jnp.float32),
          pltpu.VMEM((CT,), jnp.int32),
          pltpu.VMEM((SSROW,), jnp.int32),
          pltpu.VMEM((4096,), jnp.int32),
          pltpu.VMEM((NT, 256), jnp.int32),
          pltpu.VMEM((256,), jnp.int32),
          pltpu.VMEM((256,), jnp.int32),
          pltpu.VMEM((16,), jnp.int32),
          pltpu.VMEM((SSROW,), jnp.int32),
          pltpu.VMEM((11072,), jnp.int32),
          pltpu.VMEM((CAP,), jnp.int32),
          pltpu.VMEM((128, C), jnp.float32),
          pltpu.VMEM((128, C), jnp.float32),
          pltpu.VMEM_SHARED((4, 2, NT, 256), jnp.int32),
          pltpu.VMEM_SHARED((2, NT, 16), jnp.int32),
          pltpu.VMEM_SHARED((2, NT, SSROW), jnp.int32),
          pltpu.SemaphoreType.DMA,
          pltpu.SemaphoreType.DMA,
      ],
  )
  pooled, idx = sck(scores.reshape(B * NP), x2)
  return pooled.reshape(B, K, C), idx.reshape(B, K)


def kernel(x, W, b):
  return _run(x, W, b)
